# baseline (speedup 1.0000x reference)
import functools

import jax
import jax.numpy as jnp
from jax import lax
from jax.experimental import pallas as pl
from jax.experimental.pallas import tpu as pltpu
from jax.experimental.pallas import tpu_sc as plsc

_NC = 2
_NS = 16
_NW = _NC * _NS
_CH = 128
_NB = 4


def _sc_mesh():
  return plsc.VectorSubcoreMesh(
      core_axis_name="c", subcore_axis_name="s", num_cores=_NC,
      num_subcores=_NS)


_NB_G = 4


@functools.lru_cache(maxsize=None)
def _make_gather(n_nodes, n_edges, feat):
  n_chunks = n_edges // _CH
  per_w = (-(-n_chunks // _NW) + 7) // 8 * 8
  groups = -(-per_w // _NB_G)

  @functools.partial(
      pl.kernel,
      out_type=(
          jax.ShapeDtypeStruct((n_edges, feat), jnp.float32),
          jax.ShapeDtypeStruct((n_edges, feat), jnp.float32),
      ),
      mesh=_sc_mesh(),
      scratch_types=[
          pltpu.VMEM((per_w, _CH), jnp.int32),
          pltpu.VMEM((_NB_G, _CH, feat), jnp.float32),
          [pltpu.SemaphoreType.DMA] * _NB_G,
          [pltpu.SemaphoreType.DMA] * _NB_G,
      ],
  )
  def gather_kernel(h_hbm, src_hbm, dst_hbm, osrc_hbm, odst_hbm,
                    idx_all, rows, sg, ss):
    wid = lax.axis_index("s") * _NC + lax.axis_index("c")
    lo = wid * per_w
    n_my = jnp.minimum(per_w, n_chunks - lo)

    for idx_hbm, out_hbm in ((src_hbm, osrc_hbm), (dst_hbm, odst_hbm)):
      pltpu.sync_copy(idx_hbm.at[pl.ds(lo, per_w)], idx_all)

      def start_gather(c, b):
        pltpu.async_copy(h_hbm.at[idx_all.at[c]], rows.at[b], sg[b])

      for b in range(_NB_G):
        @pl.when(b < n_my)
        def _(b=b):
          start_gather(b, b)

      @pl.loop(0, groups)
      def _(g, out_hbm=out_hbm):
        for b in range(_NB_G):
          c = g * _NB_G + b

          @pl.when(c < n_my)
          def _(b=b, c=c, out_hbm=out_hbm):
            pltpu.make_async_copy(
                h_hbm.at[idx_all.at[c]], rows.at[b], sg[b]).wait()
            dst_slice = out_hbm.at[pl.ds((lo + c) * _CH, _CH)]
            pltpu.async_copy(rows.at[b], dst_slice, ss[b])
            nc = c + _NB_G

            @pl.when(nc < n_my)
            def _():
              pltpu.make_async_copy(rows.at[b], dst_slice, ss[b]).wait()
              start_gather(nc, b)

      for b in range(_NB_G):
        @pl.when(b < n_my)
        def _(b=b, out_hbm=out_hbm):
          pltpu.make_async_copy(
              rows.at[b], out_hbm.at[pl.ds(0, _CH)], ss[b]).wait()

  return gather_kernel


_CH_S = 128
_NB_S = 2


@functools.lru_cache(maxsize=None)
def _make_scatter(n_nodes, n_edges, feat):
  n_chunks = n_edges // _CH_S
  per_w = (-(-n_chunks // _NW) + 7) // 8 * 8
  groups = -(-per_w // _NB_S)
  out_ch = 80
  n_out_chunks = -(-n_nodes // out_ch)
  out_per_sub = -(-n_out_chunks // _NS)

  @functools.partial(
      pl.kernel,
      out_type=jax.ShapeDtypeStruct((_NC, n_nodes, feat), jnp.float32),
      mesh=_sc_mesh(),
      scratch_types=[
          pltpu.VMEM((per_w, _CH_S), jnp.int32),
          pltpu.VMEM((_NB_S, _CH_S, feat), jnp.float32),
          pltpu.VMEM_SHARED((n_nodes, feat), jnp.float32),
          [pltpu.SemaphoreType.DMA] * _NB_S,
          [pltpu.SemaphoreType.DMA] * _NB_S,
      ],
  )
  def scatter_kernel(rows_hbm, dst_hbm, zero_hbm, out_hbm,
                     idx_all, rows, acc_sh, sl, sa):
    cid = lax.axis_index("c")
    sid = lax.axis_index("s")
    wid = sid * _NC + cid
    lo = wid * per_w
    n_my = jnp.minimum(per_w, n_chunks - lo)

    @pl.loop(0, out_per_sub)
    def _(c):
      ch = sid + _NS * c

      @pl.when(ch < n_out_chunks)
      def _():
        base = ch * out_ch
        pltpu.sync_copy(zero_hbm.at[pl.ds(base, out_ch)],
                        acc_sh.at[pl.ds(base, out_ch)])

    pltpu.sync_copy(dst_hbm.at[pl.ds(lo, per_w)], idx_all)
    plsc.subcore_barrier()

    def start_load(c, b):
      pltpu.async_copy(
          rows_hbm.at[pl.ds((lo + c) * _CH_S, _CH_S)], rows.at[b], sl[b])

    for b in range(_NB_S):
      @pl.when(b < n_my)
      def _(b=b):
        start_load(b, b)

    @pl.loop(0, groups)
    def _(g):
      for b in range(_NB_S):
        c = g * _NB_S + b

        @pl.when(c < n_my)
        def _(b=b, c=c):
          pltpu.make_async_copy(
              rows_hbm.at[pl.ds(lo * _CH_S, _CH_S)], rows.at[b], sl[b]).wait()
          pltpu.async_copy(rows.at[b], acc_sh.at[idx_all.at[c]], sa[b],
                           add=True)
          nc = c + _NB_S

          @pl.when(nc < n_my)
          def _():
            pltpu.make_async_copy(
                rows.at[b], acc_sh.at[idx_all.at[0]], sa[b]).wait()
            start_load(nc, b)


    for b in range(_NB_S):
      @pl.when(b < n_my)
      def _(b=b):
        pltpu.make_async_copy(
            rows.at[b], acc_sh.at[idx_all.at[0]], sa[b]).wait()

    plsc.subcore_barrier()

    @pl.loop(0, out_per_sub)
    def _(c):
      ch = sid + _NS * c

      @pl.when(ch < n_out_chunks)
      def _():
        base = ch * out_ch
        pltpu.sync_copy(acc_sh.at[pl.ds(base, out_ch)],
                        out_hbm.at[cid, pl.ds(base, out_ch)])

  return scatter_kernel


def _mlp_body(part_sizes, has_res, out_bf16, *refs):
  n_parts = len(part_sizes)
  xs = []
  pos = 0
  for sz in part_sizes:
    if sz == 1:
      xs.append(refs[pos])
    else:
      xs.append(tuple(refs[pos:pos + sz]))
    pos += sz
  w1s = refs[pos:pos + n_parts]
  pos += n_parts
  w2, w3, w4 = refs[pos:pos + 3]
  b1, b2, b3, b4 = refs[pos + 3:pos + 7]
  pos += 7
  res = refs[pos] if has_res else None
  pos += 1 if has_res else 0
  out = refs[pos]
  out_b = refs[pos + 1] if out_bf16 else None

  def dot1(x, w):
    return jnp.dot(x, w.astype(x.dtype), preferred_element_type=jnp.float32,
                   precision=jax.lax.Precision.HIGHEST)

  def part_val(x):
    if isinstance(x, tuple):
      v = x[0][...]
      for xi in x[1:]:
        v = v + xi[...]
      return v
    return x[...]

  z = dot1(part_val(xs[0]), w1s[0][...])
  for j in range(1, n_parts):
    z += dot1(part_val(xs[j]), w1s[j][...])
  z = jnp.maximum(z + b1[...], 0.0)
  z = jnp.maximum(dot1(z, w2[...]) + b2[...], 0.0)
  z = jnp.maximum(dot1(z, w3[...]) + b3[...], 0.0)
  o = dot1(z, w4[...]) + b4[...]
  if has_res:
    o = o + res[...]
  out[...] = o
  if out_bf16:
    out_b[...] = o.astype(jnp.bfloat16)


def _mlp_call(parts, w2, w3, w4, biases, residual=None, block_rows=2000,
              out_bf16=False):
  xs_groups = [p[0] if isinstance(p[0], tuple) else (p[0],) for p in parts]
  xs = [x for g in xs_groups for x in g]
  part_sizes = tuple(len(g) for g in xs_groups)
  w1s = [p[1] for p in parts]
  m = xs[0].shape[0]
  h_out = w4.shape[1]
  grid = m // block_rows
  has_res = residual is not None

  in_specs = []
  for x in xs:
    d = x.shape[1]
    in_specs.append(pl.BlockSpec((block_rows, d), lambda i: (i, 0)))
  for w in w1s + [w2, w3, w4]:
    in_specs.append(
        pl.BlockSpec(w.shape, lambda i: (0, 0)))
  bias2d = [b.reshape(1, -1) for b in biases]
  for b in bias2d:
    in_specs.append(pl.BlockSpec(b.shape, lambda i: (0, 0)))
  args = xs + w1s + [w2, w3, w4] + bias2d
  if has_res:
    in_specs.append(pl.BlockSpec((block_rows, h_out), lambda i: (i, 0)))
    args.append(residual)

  out_spec = pl.BlockSpec((block_rows, h_out), lambda i: (i, 0))
  out_shape = jax.ShapeDtypeStruct((m, h_out), jnp.float32)
  if out_bf16:
    out_specs = (out_spec, out_spec)
    out_shapes = (out_shape, jax.ShapeDtypeStruct((m, h_out), jnp.bfloat16))
  else:
    out_specs = out_spec
    out_shapes = out_shape
  return pl.pallas_call(
      functools.partial(_mlp_body, part_sizes, has_res, out_bf16),
      grid=(grid,),
      in_specs=in_specs,
      out_specs=out_specs,
      out_shape=out_shapes,
  )(*args)


def _pool_dec_body(inv_n, *refs):
  (h, w1, w2, w3, w4, b1, b2, b3, b4, out, acc) = refs
  i = pl.program_id(0)

  @pl.when(i == 0)
  def _():
    acc[...] = jnp.zeros_like(acc)

  blk = h[...]
  acc[...] += jnp.sum(blk.reshape(-1, 8, blk.shape[1]), axis=0)

  @pl.when(i == pl.num_programs(0) - 1)
  def _():
    def dotb(a, b):
      return jnp.dot(a, b, preferred_element_type=jnp.float32,
                     precision=jax.lax.Precision.HIGHEST)

    pooled = jnp.sum(acc[...], axis=0, keepdims=True) * inv_n
    z = jnp.maximum(dotb(pooled, w1[...]) + b1[...], 0.0)
    z = jnp.maximum(dotb(z, w2[...]) + b2[...], 0.0)
    z = jnp.maximum(dotb(z, w3[...]) + b3[...], 0.0)
    out[...] = dotb(z, w4[...]) + b4[...]


def _pool_decode(h, dec_params, block_rows=2000):
  n, feat = h.shape
  grid = n // block_rows
  ws = [p["W"] for p in dec_params]
  bs = [p["b"].reshape(1, -1) for p in dec_params]
  in_specs = [pl.BlockSpec((block_rows, feat), lambda i: (i, 0))]
  for w in ws:
    in_specs.append(pl.BlockSpec(w.shape, lambda i: (0, 0)))
  for b in bs:
    in_specs.append(pl.BlockSpec(b.shape, lambda i: (0, 0)))
  out = pl.pallas_call(
      functools.partial(_pool_dec_body, 1.0 / n),
      grid=(grid,),
      in_specs=in_specs,
      out_specs=pl.BlockSpec((1, 1), lambda i: (0, 0)),
      out_shape=jax.ShapeDtypeStruct((1, 1), jnp.float32),
      scratch_shapes=[pltpu.VMEM((8, feat), jnp.float32)],
  )(h, *ws, *bs)
  return out.reshape(())


def _pad_idx(idx, n_edges, ch):
  n_chunks = n_edges // ch
  per_w = (-(-n_chunks // _NW) + 7) // 8 * 8
  total = _NW * per_w * ch
  return jnp.pad(idx, (0, total - n_edges)).reshape(-1, ch)


def _sc_gather(h, src2, dst2, n_edges):
  n, feat = h.shape
  return _make_gather(n, n_edges, feat)(h, src2, dst2)


def _sc_scatter(rows, dst2, n_nodes, zero):
  e, feat = rows.shape
  return _make_scatter(n_nodes, e, feat)(rows, dst2, zero)


def kernel(x, edge_index, edge_attr, params):
  n, feat = x.shape
  n_edges = edge_index.shape[1]
  src2 = _pad_idx(edge_index[0], n_edges, _CH)
  dst2 = _pad_idx(edge_index[1], n_edges, _CH)
  dst2s = _pad_idx(edge_index[1], n_edges, _CH_S)
  zero = jnp.zeros((n, feat), jnp.float32)

  enc_n = params["node_enc"]
  h = _mlp_call(
      [(x, enc_n[0]["W"])], enc_n[1]["W"], enc_n[2]["W"], enc_n[3]["W"],
      [p["b"] for p in enc_n])
  enc_e = params["edge_enc"]
  e = _mlp_call(
      [(edge_attr, enc_e[0]["W"])], enc_e[1]["W"], enc_e[2]["W"],
      enc_e[3]["W"], [p["b"] for p in enc_e])

  for blk in params["blocks"]:
    em = blk["edge_mlp"]
    w1 = em[0]["W"]
    h_src, h_dst = _sc_gather(h, src2, dst2, n_edges)
    e = _mlp_call(
        [(h_src, w1[:feat]), (h_dst, w1[feat:2 * feat]), (e, w1[2 * feat:])],
        em[1]["W"], em[2]["W"], em[3]["W"], [p["b"] for p in em],
        residual=e)
    agg = _sc_scatter(e, dst2s, n, zero)
    nm = blk["node_mlp"]
    nw1 = nm[0]["W"]
    h = _mlp_call(
        [(h, nw1[:feat]), ((agg[0], agg[1]), nw1[feat:])],
        nm[1]["W"], nm[2]["W"], nm[3]["W"], [p["b"] for p in nm],
        residual=h)

  return _pool_decode(h, params["decoder"])


# speedup vs baseline: 2.7192x; 2.7192x over previous
import functools

import jax
import jax.numpy as jnp
from jax import lax
from jax.experimental import pallas as pl
from jax.experimental.pallas import tpu as pltpu
from jax.experimental.pallas import tpu_sc as plsc

_NC = 2
_NS = 16
_NW = _NC * _NS
_CH = 128
_NB = 4


def _sc_mesh():
  return plsc.VectorSubcoreMesh(
      core_axis_name="c", subcore_axis_name="s", num_cores=_NC,
      num_subcores=_NS)


_NB_G = 4


@functools.lru_cache(maxsize=None)
def _make_gather(n_nodes, n_edges, feat):
  n_chunks = n_edges // _CH
  per_w = (-(-n_chunks // _NW) + 7) // 8 * 8
  groups = -(-per_w // _NB_G)

  @functools.partial(
      pl.kernel,
      out_type=(
          jax.ShapeDtypeStruct((n_edges, feat), jnp.float32),
          jax.ShapeDtypeStruct((n_edges, feat), jnp.float32),
      ),
      mesh=_sc_mesh(),
      scratch_types=[
          pltpu.VMEM((per_w, _CH), jnp.int32),
          pltpu.VMEM((_NB_G, _CH, feat), jnp.float32),
          [pltpu.SemaphoreType.DMA] * _NB_G,
          [pltpu.SemaphoreType.DMA] * _NB_G,
      ],
  )
  def gather_kernel(h_hbm, src_hbm, dst_hbm, osrc_hbm, odst_hbm,
                    idx_all, rows, sg, ss):
    wid = lax.axis_index("s") * _NC + lax.axis_index("c")
    lo = wid * per_w
    n_my = jnp.minimum(per_w, n_chunks - lo)

    for idx_hbm, out_hbm in ((src_hbm, osrc_hbm), (dst_hbm, odst_hbm)):
      pltpu.sync_copy(idx_hbm.at[pl.ds(lo, per_w)], idx_all)

      def start_gather(c, b):
        pltpu.async_copy(h_hbm.at[idx_all.at[c]], rows.at[b], sg[b])

      for b in range(_NB_G):
        @pl.when(b < n_my)
        def _(b=b):
          start_gather(b, b)

      @pl.loop(0, groups)
      def _(g, out_hbm=out_hbm):
        for b in range(_NB_G):
          c = g * _NB_G + b

          @pl.when(c < n_my)
          def _(b=b, c=c, out_hbm=out_hbm):
            pltpu.make_async_copy(
                h_hbm.at[idx_all.at[c]], rows.at[b], sg[b]).wait()
            dst_slice = out_hbm.at[pl.ds((lo + c) * _CH, _CH)]
            pltpu.async_copy(rows.at[b], dst_slice, ss[b])
            nc = c + _NB_G

            @pl.when(nc < n_my)
            def _():
              pltpu.make_async_copy(rows.at[b], dst_slice, ss[b]).wait()
              start_gather(nc, b)

      for b in range(_NB_G):
        @pl.when(b < n_my)
        def _(b=b, out_hbm=out_hbm):
          pltpu.make_async_copy(
              rows.at[b], out_hbm.at[pl.ds(0, _CH)], ss[b]).wait()

  return gather_kernel


_CH_S = 128
_NB_S = 2


@functools.lru_cache(maxsize=None)
def _make_scatter(n_nodes, n_edges, feat):
  n_chunks = n_edges // _CH_S
  per_w = (-(-n_chunks // _NW) + 7) // 8 * 8
  groups = -(-per_w // _NB_S)
  out_ch = 80
  n_out_chunks = -(-n_nodes // out_ch)
  out_per_sub = -(-n_out_chunks // _NS)

  @functools.partial(
      pl.kernel,
      out_type=jax.ShapeDtypeStruct((_NC, n_nodes, feat), jnp.float32),
      mesh=_sc_mesh(),
      scratch_types=[
          pltpu.VMEM((per_w, _CH_S), jnp.int32),
          pltpu.VMEM((_NB_S, _CH_S, feat), jnp.float32),
          pltpu.VMEM_SHARED((n_nodes, feat), jnp.float32),
          [pltpu.SemaphoreType.DMA] * _NB_S,
          [pltpu.SemaphoreType.DMA] * _NB_S,
      ],
  )
  def scatter_kernel(rows_hbm, dst_hbm, zero_hbm, out_hbm,
                     idx_all, rows, acc_sh, sl, sa):
    cid = lax.axis_index("c")
    sid = lax.axis_index("s")
    wid = sid * _NC + cid
    lo = wid * per_w
    n_my = jnp.minimum(per_w, n_chunks - lo)

    @pl.loop(0, out_per_sub)
    def _(c):
      ch = sid + _NS * c

      @pl.when(ch < n_out_chunks)
      def _():
        base = ch * out_ch
        pltpu.sync_copy(zero_hbm.at[pl.ds(base, out_ch)],
                        acc_sh.at[pl.ds(base, out_ch)])

    pltpu.sync_copy(dst_hbm.at[pl.ds(lo, per_w)], idx_all)
    plsc.subcore_barrier()

    def start_load(c, b):
      pltpu.async_copy(
          rows_hbm.at[pl.ds((lo + c) * _CH_S, _CH_S)], rows.at[b], sl[b])

    for b in range(_NB_S):
      @pl.when(b < n_my)
      def _(b=b):
        start_load(b, b)

    @pl.loop(0, groups)
    def _(g):
      for b in range(_NB_S):
        c = g * _NB_S + b

        @pl.when(c < n_my)
        def _(b=b, c=c):
          pltpu.make_async_copy(
              rows_hbm.at[pl.ds(lo * _CH_S, _CH_S)], rows.at[b], sl[b]).wait()
          pltpu.async_copy(rows.at[b], acc_sh.at[idx_all.at[c]], sa[b],
                           add=True)
          nc = c + _NB_S

          @pl.when(nc < n_my)
          def _():
            pltpu.make_async_copy(
                rows.at[b], acc_sh.at[idx_all.at[0]], sa[b]).wait()
            start_load(nc, b)


    for b in range(_NB_S):
      @pl.when(b < n_my)
      def _(b=b):
        pltpu.make_async_copy(
            rows.at[b], acc_sh.at[idx_all.at[0]], sa[b]).wait()

    plsc.subcore_barrier()

    @pl.loop(0, out_per_sub)
    def _(c):
      ch = sid + _NS * c

      @pl.when(ch < n_out_chunks)
      def _():
        base = ch * out_ch
        pltpu.sync_copy(acc_sh.at[pl.ds(base, out_ch)],
                        out_hbm.at[cid, pl.ds(base, out_ch)])

  return scatter_kernel


def _mlp_body(part_sizes, has_res, out_bf16, *refs):
  n_parts = len(part_sizes)
  xs = []
  pos = 0
  for sz in part_sizes:
    if sz == 1:
      xs.append(refs[pos])
    else:
      xs.append(tuple(refs[pos:pos + sz]))
    pos += sz
  w1s = refs[pos:pos + n_parts]
  pos += n_parts
  w2, w3, w4 = refs[pos:pos + 3]
  b1, b2, b3, b4 = refs[pos + 3:pos + 7]
  pos += 7
  res = refs[pos] if has_res else None
  pos += 1 if has_res else 0
  out = refs[pos]
  out_b = refs[pos + 1] if out_bf16 else None

  def dot1(x, w):
    return jnp.dot(x, w.astype(x.dtype), preferred_element_type=jnp.float32)

  def part_val(x):
    if isinstance(x, tuple):
      v = x[0][...]
      for xi in x[1:]:
        v = v + xi[...]
      return v
    return x[...]

  z = dot1(part_val(xs[0]), w1s[0][...])
  for j in range(1, n_parts):
    z += dot1(part_val(xs[j]), w1s[j][...])
  z = jnp.maximum(z + b1[...], 0.0)
  z = jnp.maximum(dot1(z, w2[...]) + b2[...], 0.0)
  z = jnp.maximum(dot1(z, w3[...]) + b3[...], 0.0)
  o = dot1(z, w4[...]) + b4[...]
  if has_res:
    o = o + res[...]
  out[...] = o
  if out_bf16:
    out_b[...] = o.astype(jnp.bfloat16)


def _mlp_call(parts, w2, w3, w4, biases, residual=None, block_rows=2000,
              out_bf16=False):
  xs_groups = [p[0] if isinstance(p[0], tuple) else (p[0],) for p in parts]
  xs = [x for g in xs_groups for x in g]
  part_sizes = tuple(len(g) for g in xs_groups)
  w1s = [p[1] for p in parts]
  m = xs[0].shape[0]
  h_out = w4.shape[1]
  grid = m // block_rows
  has_res = residual is not None

  in_specs = []
  for x in xs:
    d = x.shape[1]
    in_specs.append(pl.BlockSpec((block_rows, d), lambda i: (i, 0)))
  for w in w1s + [w2, w3, w4]:
    in_specs.append(
        pl.BlockSpec(w.shape, lambda i: (0, 0)))
  bias2d = [b.reshape(1, -1) for b in biases]
  for b in bias2d:
    in_specs.append(pl.BlockSpec(b.shape, lambda i: (0, 0)))
  args = xs + w1s + [w2, w3, w4] + bias2d
  if has_res:
    in_specs.append(pl.BlockSpec((block_rows, h_out), lambda i: (i, 0)))
    args.append(residual)

  out_spec = pl.BlockSpec((block_rows, h_out), lambda i: (i, 0))
  out_shape = jax.ShapeDtypeStruct((m, h_out), jnp.float32)
  if out_bf16:
    out_specs = (out_spec, out_spec)
    out_shapes = (out_shape, jax.ShapeDtypeStruct((m, h_out), jnp.bfloat16))
  else:
    out_specs = out_spec
    out_shapes = out_shape
  return pl.pallas_call(
      functools.partial(_mlp_body, part_sizes, has_res, out_bf16),
      grid=(grid,),
      in_specs=in_specs,
      out_specs=out_specs,
      out_shape=out_shapes,
  )(*args)


def _pool_dec_body(inv_n, *refs):
  (h, w1, w2, w3, w4, b1, b2, b3, b4, out, acc) = refs
  i = pl.program_id(0)

  @pl.when(i == 0)
  def _():
    acc[...] = jnp.zeros_like(acc)

  blk = h[...]
  acc[...] += jnp.sum(blk.reshape(-1, 8, blk.shape[1]), axis=0)

  @pl.when(i == pl.num_programs(0) - 1)
  def _():
    def dotb(a, b):
      return jnp.dot(a, b, preferred_element_type=jnp.float32)

    pooled = jnp.sum(acc[...], axis=0, keepdims=True) * inv_n
    z = jnp.maximum(dotb(pooled, w1[...]) + b1[...], 0.0)
    z = jnp.maximum(dotb(z, w2[...]) + b2[...], 0.0)
    z = jnp.maximum(dotb(z, w3[...]) + b3[...], 0.0)
    out[...] = dotb(z, w4[...]) + b4[...]


def _pool_decode(h, dec_params, block_rows=2000):
  n, feat = h.shape
  grid = n // block_rows
  ws = [p["W"] for p in dec_params]
  bs = [p["b"].reshape(1, -1) for p in dec_params]
  in_specs = [pl.BlockSpec((block_rows, feat), lambda i: (i, 0))]
  for w in ws:
    in_specs.append(pl.BlockSpec(w.shape, lambda i: (0, 0)))
  for b in bs:
    in_specs.append(pl.BlockSpec(b.shape, lambda i: (0, 0)))
  out = pl.pallas_call(
      functools.partial(_pool_dec_body, 1.0 / n),
      grid=(grid,),
      in_specs=in_specs,
      out_specs=pl.BlockSpec((1, 1), lambda i: (0, 0)),
      out_shape=jax.ShapeDtypeStruct((1, 1), jnp.float32),
      scratch_shapes=[pltpu.VMEM((8, feat), jnp.float32)],
  )(h, *ws, *bs)
  return out.reshape(())


def _pad_idx(idx, n_edges, ch):
  n_chunks = n_edges // ch
  per_w = (-(-n_chunks // _NW) + 7) // 8 * 8
  total = _NW * per_w * ch
  return jnp.pad(idx, (0, total - n_edges)).reshape(-1, ch)


def _sc_gather(h, src2, dst2, n_edges):
  n, feat = h.shape
  return _make_gather(n, n_edges, feat)(h, src2, dst2)


def _sc_scatter(rows, dst2, n_nodes, zero):
  e, feat = rows.shape
  return _make_scatter(n_nodes, e, feat)(rows, dst2, zero)


def kernel(x, edge_index, edge_attr, params):
  n, feat = x.shape
  n_edges = edge_index.shape[1]
  src2 = _pad_idx(edge_index[0], n_edges, _CH)
  dst2 = _pad_idx(edge_index[1], n_edges, _CH)
  dst2s = _pad_idx(edge_index[1], n_edges, _CH_S)
  zero = jnp.zeros((n, feat), jnp.float32)

  enc_n = params["node_enc"]
  h = _mlp_call(
      [(x, enc_n[0]["W"])], enc_n[1]["W"], enc_n[2]["W"], enc_n[3]["W"],
      [p["b"] for p in enc_n])
  enc_e = params["edge_enc"]
  e = _mlp_call(
      [(edge_attr, enc_e[0]["W"])], enc_e[1]["W"], enc_e[2]["W"],
      enc_e[3]["W"], [p["b"] for p in enc_e])

  for blk in params["blocks"]:
    em = blk["edge_mlp"]
    w1 = em[0]["W"]
    h_src, h_dst = _sc_gather(h, src2, dst2, n_edges)
    e = _mlp_call(
        [(h_src, w1[:feat]), (h_dst, w1[feat:2 * feat]), (e, w1[2 * feat:])],
        em[1]["W"], em[2]["W"], em[3]["W"], [p["b"] for p in em],
        residual=e)
    agg = _sc_scatter(e, dst2s, n, zero)
    nm = blk["node_mlp"]
    nw1 = nm[0]["W"]
    h = _mlp_call(
        [(h, nw1[:feat]), ((agg[0], agg[1]), nw1[feat:])],
        nm[1]["W"], nm[2]["W"], nm[3]["W"], [p["b"] for p in nm],
        residual=h)

  return _pool_decode(h, params["decoder"])
